# baseline (speedup 1.0000x reference)
import functools
import math

import jax
import jax.numpy as jnp
import numpy as np
from jax import lax
from jax.experimental import pallas as pl
from jax.experimental.pallas import tpu as pltpu
from jax.experimental.pallas import tpu_sc as plsc

HIDDEN = 1024
NUM_FEATURES = 8
B, S = 4, 4096
N_TOK = B * S

NSLICE = 4
SLICE_TOK = N_TOK // NSLICE

NC, NS = 2, 16
NW = NC * NS
ROWS_PER_W = SLICE_TOK // NW
CHUNK = 32
NCHUNK = ROWS_PER_W // CHUNK

TOK_BLK = 512
BLK_PER_SLICE = SLICE_TOK // TOK_BLK
SMALL_PAD = 256


def _sc_gather(table, idx3):
  mesh = plsc.VectorSubcoreMesh(core_axis_name="c", subcore_axis_name="s")

  @functools.partial(
      pl.kernel,
      out_type=jax.ShapeDtypeStruct((SLICE_TOK, HIDDEN), jnp.float32),
      mesh=mesh,
      scratch_types=[
          pltpu.VMEM((NCHUNK, CHUNK), jnp.int32),
          pltpu.VMEM((CHUNK, HIDDEN), jnp.float32),
          pltpu.VMEM((CHUNK, HIDDEN), jnp.float32),
          pltpu.VMEM((CHUNK, HIDDEN), jnp.float32),
          pltpu.SemaphoreType.DMA,
          pltpu.SemaphoreType.DMA,
          pltpu.SemaphoreType.DMA,
      ],
  )
  def k(table_hbm, idx_hbm, out_hbm, idx_v, buf0, buf1, buf2,
        sem0, sem1, sem2):
    wid = lax.axis_index("s") * NC + lax.axis_index("c")
    base = wid * ROWS_PER_W
    pltpu.sync_copy(idx_hbm.at[wid], idx_v)
    bufs = (buf0, buf1, buf2)
    sems = (sem0, sem1, sem2)
    copies = [None, None, None]
    copies[0] = pltpu.async_copy(table_hbm.at[idx_v.at[0]], bufs[0], sems[0])
    copies[1] = pltpu.async_copy(table_hbm.at[idx_v.at[1]], bufs[1], sems[1])
    for c in range(2, NCHUNK):
      copies[c % 3] = pltpu.async_copy(
          table_hbm.at[idx_v.at[c]], bufs[c % 3], sems[c % 3])
      copies[(c - 2) % 3].wait()
      pltpu.sync_copy(bufs[(c - 2) % 3],
                      out_hbm.at[pl.ds(base + (c - 2) * CHUNK, CHUNK)])
    for c in range(NCHUNK - 2, NCHUNK):
      copies[c % 3].wait()
      pltpu.sync_copy(bufs[c % 3],
                      out_hbm.at[pl.ds(base + c * CHUNK, CHUNK)])

  return k(table, idx3)


_SIN_C = (6.2794107, -41.1184955, 78.1214006, -56.4957497)
_COS_C = (0.998808134, -19.5698817, 61.2749502, -59.9673766)
_NCOEF = 4


def _tc_fuse_body(*refs):
  (gath_ref, vidx_ref, ridx_ref, lidx_ref, stf_ref, sm_ref, d2_ref,
   coef_ref, out_ref) = refs[-9:]
  v = vidx_ref[...]
  r = ridx_ref[...] + 128
  l = lidx_ref[...] + 144
  iot = lax.broadcasted_iota(jnp.int32, (TOK_BLK, SMALL_PAD), 1)
  oh = ((iot == v) | (iot == r) | (iot == l)).astype(jnp.float32)
  small = jnp.dot(oh, sm_ref[...], preferred_element_type=jnp.float32)
  stf = stf_ref[...]
  d2b = d2_ref[...]
  cf = coef_ref[...]
  a0 = cf[0:1, :]
  a1 = cf[1:2, :]
  a2 = cf[2:3, :]
  a3 = cf[3:4, :]
  ev = cf[4:5, :]
  od = 1.0 - ev
  acc_a = None
  acc_b = None
  for f in range(NUM_FEATURES):
    q = stf[:, f:f + 1] * d2b
    fr = q - jnp.round(q)
    u = fr * fr
    p = ((a3 * u + a2) * u + a1) * u + a0
    if acc_a is None:
      acc_a = p
      acc_b = p * fr
    else:
      acc_a = acc_a + p
      acc_b = p * fr + acc_b
  enc = acc_b * ev + acc_a * od
  out_ref[...] = gath_ref[...] + small + enc


def _tc_fuse_slice(s, buf, gathered, var_idx, res_idx, leadtime_idx, stf,
                   sm_table, d2, coef):
  off = s * BLK_PER_SLICE

  def out_map(i, off=off):
    return (off + i, 0)

  in_specs = [
      pl.BlockSpec((TOK_BLK, HIDDEN), lambda i: (i, 0)),
      pl.BlockSpec((TOK_BLK, 1), lambda i: (i, 0)),
      pl.BlockSpec((TOK_BLK, 1), lambda i: (i, 0)),
      pl.BlockSpec((TOK_BLK, 1), lambda i: (i, 0)),
      pl.BlockSpec((TOK_BLK, NUM_FEATURES), lambda i: (i, 0)),
      pl.BlockSpec((SMALL_PAD, HIDDEN), lambda i: (0, 0)),
      pl.BlockSpec((1, HIDDEN), lambda i: (0, 0)),
      pl.BlockSpec((5, HIDDEN), lambda i: (0, 0)),
  ]
  args = (gathered, var_idx, res_idx, leadtime_idx, stf, sm_table, d2, coef)
  aliases = {}
  if buf is not None:
    in_specs = [pl.BlockSpec(memory_space=pl.ANY)] + in_specs
    args = (buf,) + args
    aliases = {0: 0}
  return pl.pallas_call(
      _tc_fuse_body,
      grid=(BLK_PER_SLICE,),
      in_specs=in_specs,
      out_specs=pl.BlockSpec((TOK_BLK, HIDDEN), out_map),
      out_shape=jax.ShapeDtypeStruct((N_TOK, HIDDEN), jnp.float32),
      input_output_aliases=aliases,
  )(*args)


def kernel(input_ids, position_ids, var_idx, res_idx, leadtime_idx,
           spatial_temporal_features, token_table, var_table, res_table,
           leadtime_table):
  ids4 = input_ids.astype(jnp.int32).reshape(NSLICE, NW, NCHUNK, CHUNK)

  sm_table = jnp.concatenate(
      [var_table, res_table, leadtime_table,
       jnp.zeros((SMALL_PAD - 208, HIDDEN), jnp.float32)], axis=0)

  div_term = np.exp(np.arange(0, HIDDEN, 2, dtype=np.float32)
                    * (-math.log(10000.0) / HIDDEN))
  d2b = np.repeat(div_term * (2048.0 / (2.0 * math.pi)), 2)
  d2b = d2b.reshape(1, HIDDEN).astype(np.float32)

  coef = np.zeros((_NCOEF + 1, HIDDEN), np.float32)
  for i in range(_NCOEF):
    coef[i, 0::2] = _SIN_C[i] / NUM_FEATURES
    coef[i, 1::2] = _COS_C[i] / NUM_FEATURES
  coef[_NCOEF, 0::2] = 1.0
  coef_j = jnp.asarray(coef)

  vi = var_idx.astype(jnp.int32).reshape(NSLICE, SLICE_TOK, 1)
  ri = res_idx.astype(jnp.int32).reshape(NSLICE, SLICE_TOK, 1)
  li = leadtime_idx.astype(jnp.int32).reshape(NSLICE, SLICE_TOK, 1)
  st = spatial_temporal_features.reshape(NSLICE, SLICE_TOK, NUM_FEATURES)
  d2b_j = jnp.asarray(d2b)

  g = [_sc_gather(token_table, ids4[s]) for s in range(NSLICE)]
  buf = None
  for s in range(NSLICE):
    buf = _tc_fuse_slice(s, buf, g[s], vi[s], ri[s], li[s], st[s],
                         sm_table, d2b_j, coef_j)
  return (buf, position_ids)


# speedup vs baseline: 1.0133x; 1.0133x over previous
import functools
import math

import jax
import jax.numpy as jnp
import numpy as np
from jax import lax
from jax.experimental import pallas as pl
from jax.experimental.pallas import tpu as pltpu
from jax.experimental.pallas import tpu_sc as plsc

HIDDEN = 1024
NUM_FEATURES = 8
B, S = 4, 4096
N_TOK = B * S

NC, NS = 2, 16
NW = NC * NS
ROWS_PER_W = N_TOK // NW
CHUNK = 32
NCHUNK = ROWS_PER_W // CHUNK

TOK_BLK = 1024
N_BLK = N_TOK // TOK_BLK
SMALL_PAD = 256


def _sc_gather(table, idx3):
  mesh = plsc.VectorSubcoreMesh(core_axis_name="c", subcore_axis_name="s")

  @functools.partial(
      pl.kernel,
      out_type=jax.ShapeDtypeStruct((N_TOK, HIDDEN), jnp.float32),
      mesh=mesh,
      scratch_types=[
          pltpu.VMEM((NCHUNK, CHUNK), jnp.int32),
          pltpu.VMEM((CHUNK, HIDDEN), jnp.float32),
          pltpu.VMEM((CHUNK, HIDDEN), jnp.float32),
          pltpu.SemaphoreType.DMA,
          pltpu.SemaphoreType.DMA,
      ],
  )
  def k(table_hbm, idx_hbm, out_hbm, idx_v, buf0, buf1, sem0, sem1):
    wid = lax.axis_index("s") * NC + lax.axis_index("c")
    base = wid * ROWS_PER_W
    pltpu.sync_copy(idx_hbm.at[wid], idx_v)
    bufs = (buf0, buf1)
    sems = (sem0, sem1)
    copies = [None, None]
    copies[0] = pltpu.async_copy(table_hbm.at[idx_v.at[0]], buf0, sem0)
    for c in range(1, NCHUNK):
      copies[c % 2] = pltpu.async_copy(
          table_hbm.at[idx_v.at[c]], bufs[c % 2], sems[c % 2])
      copies[(c - 1) % 2].wait()
      pltpu.sync_copy(bufs[(c - 1) % 2],
                      out_hbm.at[pl.ds(base + (c - 1) * CHUNK, CHUNK)])
    last = NCHUNK - 1
    copies[last % 2].wait()
    pltpu.sync_copy(bufs[last % 2],
                    out_hbm.at[pl.ds(base + last * CHUNK, CHUNK)])

  return k(table, idx3)


_SIN_C = (6.2794107, -41.1184955, 78.1214006, -56.4957497)
_COS_C = (0.998808134, -19.5698817, 61.2749502, -59.9673766)
_NCOEF = 4


def _tc_fuse_body(gath_ref, vidx_ref, ridx_ref, lidx_ref, stf_ref, sm_ref,
                  d2_ref, coef_ref, out_ref):
  v = vidx_ref[...]
  r = ridx_ref[...] + 128
  l = lidx_ref[...] + 144
  iot = lax.broadcasted_iota(jnp.int32, (TOK_BLK, SMALL_PAD), 1)
  oh = ((iot == v) | (iot == r) | (iot == l)).astype(jnp.float32)
  small = jnp.dot(oh, sm_ref[...], preferred_element_type=jnp.float32)
  stf = stf_ref[...]
  d2b = d2_ref[...]
  cf = coef_ref[...]
  a0 = cf[0:1, :]
  a1 = cf[1:2, :]
  a2 = cf[2:3, :]
  a3 = cf[3:4, :]
  ev = cf[4:5, :]
  od = 1.0 - ev
  acc_a = None
  acc_b = None
  for f in range(NUM_FEATURES):
    q = stf[:, f:f + 1] * d2b
    fr = q - jnp.round(q)
    u = fr * fr
    p = ((a3 * u + a2) * u + a1) * u + a0
    if acc_a is None:
      acc_a = p
      acc_b = p * fr
    else:
      acc_a = acc_a + p
      acc_b = p * fr + acc_b
  enc = acc_b * ev + acc_a * od
  out_ref[...] = gath_ref[...] + small + enc


def _tc_fuse(gathered, var_idx, res_idx, leadtime_idx, stf, sm_table, d2, coef):
  grid = (N_BLK,)
  return pl.pallas_call(
      _tc_fuse_body,
      grid=grid,
      in_specs=[
          pl.BlockSpec((TOK_BLK, HIDDEN), lambda i: (i, 0)),
          pl.BlockSpec((TOK_BLK, 1), lambda i: (i, 0)),
          pl.BlockSpec((TOK_BLK, 1), lambda i: (i, 0)),
          pl.BlockSpec((TOK_BLK, 1), lambda i: (i, 0)),
          pl.BlockSpec((TOK_BLK, NUM_FEATURES), lambda i: (i, 0)),
          pl.BlockSpec((SMALL_PAD, HIDDEN), lambda i: (0, 0)),
          pl.BlockSpec((1, HIDDEN), lambda i: (0, 0)),
          pl.BlockSpec((5, HIDDEN), lambda i: (0, 0)),
      ],
      out_specs=pl.BlockSpec((TOK_BLK, HIDDEN), lambda i: (i, 0)),
      out_shape=jax.ShapeDtypeStruct((N_TOK, HIDDEN), jnp.float32),
  )(gathered, var_idx, res_idx, leadtime_idx, stf, sm_table, d2, coef)


def kernel(input_ids, position_ids, var_idx, res_idx, leadtime_idx,
           spatial_temporal_features, token_table, var_table, res_table,
           leadtime_table):
  ids3 = input_ids.astype(jnp.int32).reshape(NW, NCHUNK, CHUNK)
  gathered = _sc_gather(token_table, ids3)

  sm_table = jnp.concatenate(
      [var_table, res_table, leadtime_table,
       jnp.zeros((SMALL_PAD - 208, HIDDEN), jnp.float32)], axis=0)

  div_term = np.exp(np.arange(0, HIDDEN, 2, dtype=np.float32)
                    * (-math.log(10000.0) / HIDDEN))
  d2b = np.repeat(div_term * (2048.0 / (2.0 * math.pi)), 2)
  d2b = d2b.reshape(1, HIDDEN).astype(np.float32)

  coef = np.zeros((_NCOEF + 1, HIDDEN), np.float32)
  for i in range(_NCOEF):
    coef[i, 0::2] = _SIN_C[i] / NUM_FEATURES
    coef[i, 1::2] = _COS_C[i] / NUM_FEATURES
  coef[_NCOEF, 0::2] = 1.0
  coef_j = jnp.asarray(coef)

  out = _tc_fuse(
      gathered,
      var_idx.astype(jnp.int32).reshape(N_TOK, 1),
      res_idx.astype(jnp.int32).reshape(N_TOK, 1),
      leadtime_idx.astype(jnp.int32).reshape(N_TOK, 1),
      spatial_temporal_features.reshape(N_TOK, NUM_FEATURES),
      sm_table, jnp.asarray(d2b), coef_j)
  return (out, position_ids)
